# baseline (speedup 1.0000x reference)
import jax
import jax.numpy as jnp
from jax import lax
from jax.experimental import pallas as pl
from jax.experimental.pallas import tpu as pltpu

BM = 1024
SUB = 256
LANES = 128
NCH = SUB // LANES


def _argmin_kernel(x_ref, c_ref, cn_ref, out_ref):
    K = c_ref.shape[1]
    nsub = K // SUB

    xb = x_ref[...]
    xsq = jnp.sum(xb * xb, axis=1, keepdims=True)
    xb2h = (xb * -2.0).astype(jnp.bfloat16)

    m = None
    a = None
    for n in range(nsub):
        acc2 = jnp.dot(
            xb2h,
            c_ref[:, n * SUB:(n + 1) * SUB].astype(jnp.bfloat16),
            preferred_element_type=jnp.float32,
        )
        for t in range(NCH):
            k = n * NCH + t
            sl = slice(t * LANES, (t + 1) * LANES)
            ch = (xsq + acc2[:, sl]) + cn_ref[:, k * LANES:(k + 1) * LANES]
            if m is None:
                m, a = ch, jnp.zeros_like(ch)
            else:
                upd = ch < m
                m = jnp.minimum(m, ch)
                a = jnp.where(upd, float(k), a)

    lane = lax.broadcasted_iota(jnp.int32, m.shape, 1).astype(jnp.float32)
    col = a * float(LANES) + lane
    bv = jnp.min(m, axis=1, keepdims=True)
    bi = jnp.min(jnp.where(m == bv, col, float(K)), axis=1, keepdims=True)
    out_ref[...] = bi.astype(jnp.int32)


def kernel(x, C, Cnorm):
    B, T, D = x.shape
    K = C.shape[1]
    M = B * T
    x2 = x.reshape(M, D)

    out = pl.pallas_call(
        _argmin_kernel,
        grid=(M // BM,),
        in_specs=[
            pl.BlockSpec((BM, D), lambda i: (i, 0)),
            pl.BlockSpec((D, K), lambda i: (0, 0)),
            pl.BlockSpec((1, K), lambda i: (0, 0)),
        ],
        out_specs=pl.BlockSpec((BM, 1), lambda i: (i, 0)),
        out_shape=jax.ShapeDtypeStruct((M, 1), jnp.int32),
        compiler_params=pltpu.CompilerParams(
            dimension_semantics=("arbitrary",),
            vmem_limit_bytes=60000 * 1024,
        ),
    )(x2, C, Cnorm)
    return out.reshape(B, T, 1)
